# baseline (speedup 1.0000x reference)
import jax
import jax.numpy as jnp
from jax import lax
from jax.experimental import pallas as pl
from jax.experimental.pallas import tpu as pltpu
from jax.experimental.pallas import tpu_sc as plsc

_CONTEXT_WINDOW = 4096


def _emb_gather_sc(emb_hbm, tok_hbm, out_hbm, idx_v, row_v, sem):
    cid = lax.axis_index("c")
    sid = lax.axis_index("s")

    @pl.when((cid == 0) & (sid == 0))
    def _():
        pltpu.sync_copy(tok_hbm, idx_v)
        pltpu.async_copy(emb_hbm.at[idx_v], row_v, sem).wait()
        pltpu.sync_copy(row_v, out_hbm)


def _emb_row_sc(emb, token, d):
    tok8 = jnp.full((8,), token, jnp.int32)
    mesh = plsc.VectorSubcoreMesh(core_axis_name="c", subcore_axis_name="s")
    return pl.kernel(
        _emb_gather_sc,
        out_type=jax.ShapeDtypeStruct((8, d), emb.dtype),
        mesh=mesh,
        scratch_types=[
            pltpu.VMEM((8,), jnp.int32),
            pltpu.VMEM((8, d), emb.dtype),
            pltpu.SemaphoreType.DMA,
        ],
    )(emb, tok8)


def _shift_copy_kernel(idea_ref, out_ref, carry_ref):
    r = idea_ref.shape[1]
    out_ref[0, 0:r - 1, :] = idea_ref[0, 1:r, :]
    out_ref[0, r - 1:r, :] = carry_ref[...]
    carry_ref[...] = idea_ref[0, 0:1, :]


def _combine_kernel(row_ref, out0_ref, out_ref, tail_v, sem, *, b, l):
    del out0_ref
    for bb in range(b):
        pltpu.make_async_copy(
            out_ref.at[bb, pl.ds(l - 8, 8)], tail_v, sem).start()
        pltpu.make_async_copy(
            out_ref.at[bb, pl.ds(l - 8, 8)], tail_v, sem).wait()
        tail_v[7:8, :] = row_ref[0:1, :]
        pltpu.make_async_copy(
            tail_v, out_ref.at[bb, pl.ds(l - 8, 8)], sem).start()
        pltpu.make_async_copy(
            tail_v, out_ref.at[bb, pl.ds(l - 8, 8)], sem).wait()


def kernel(idea, token, emb):
    import functools

    b, l, d = idea.shape
    lout = min(_CONTEXT_WINDOW, l + 1)
    if lout == l + 1:
        idea = jnp.concatenate([jnp.zeros((b, 1, d), idea.dtype), idea], axis=1)
        l = lout
    r = 1024 if l % 1024 == 0 else l
    nb = l // r
    emb_row = _emb_row_sc(emb, token, d)
    out0 = pl.pallas_call(
        _shift_copy_kernel,
        grid=(b, nb),
        in_specs=[pl.BlockSpec((1, r, d), lambda bb, j: (bb, nb - 1 - j, 0))],
        out_specs=pl.BlockSpec((1, r, d), lambda bb, j: (bb, nb - 1 - j, 0)),
        scratch_shapes=[pltpu.VMEM((1, d), idea.dtype)],
        out_shape=jax.ShapeDtypeStruct((b, l, d), idea.dtype),
        compiler_params=pltpu.CompilerParams(
            dimension_semantics=("parallel", "arbitrary"),
            vmem_limit_bytes=100 * 1024 * 1024,
        ),
    )(idea)
    out = pl.pallas_call(
        functools.partial(_combine_kernel, b=b, l=l),
        in_specs=[
            pl.BlockSpec(memory_space=pltpu.MemorySpace.VMEM),
            pl.BlockSpec(memory_space=pl.ANY),
        ],
        out_specs=pl.BlockSpec(memory_space=pl.ANY),
        scratch_shapes=[pltpu.VMEM((8, d), idea.dtype),
                        pltpu.SemaphoreType.DMA],
        out_shape=jax.ShapeDtypeStruct((b, l, d), idea.dtype),
        input_output_aliases={1: 0},
    )(emb_row, out0)
    return out


# speedup vs baseline: 1.0049x; 1.0049x over previous
import jax
import jax.numpy as jnp
from jax import lax
from jax.experimental import pallas as pl
from jax.experimental.pallas import tpu as pltpu
from jax.experimental.pallas import tpu_sc as plsc

_CONTEXT_WINDOW = 4096


def _emb_gather_sc(emb_hbm, tok_hbm, out_hbm, tok_s, row_sh, sem):
    cid = lax.axis_index("c")

    @pl.when(cid == 0)
    def _():
        pltpu.sync_copy(tok_hbm, tok_s)
        t = tok_s[0]
        pltpu.async_copy(emb_hbm.at[pl.ds(t, 1)], row_sh, sem).wait()
        pltpu.sync_copy(row_sh, out_hbm.at[pl.ds(0, 1)])


def _emb_row_sc(emb, token, d):
    tok1 = jnp.full((1,), token, jnp.int32)
    mesh = plsc.ScalarSubcoreMesh(axis_name="c", num_cores=2)
    return pl.kernel(
        _emb_gather_sc,
        out_type=jax.ShapeDtypeStruct((8, d), emb.dtype),
        mesh=mesh,
        scratch_types=[
            pltpu.SMEM((1,), jnp.int32),
            pltpu.VMEM_SHARED((1, d), emb.dtype),
            pltpu.SemaphoreType.DMA,
        ],
    )(emb, tok1)


def _shift_copy_kernel(idea_ref, out_ref, carry_ref):
    r = idea_ref.shape[1]
    out_ref[0, 0:r - 1, :] = idea_ref[0, 1:r, :]
    out_ref[0, r - 1:r, :] = carry_ref[...]
    carry_ref[...] = idea_ref[0, 0:1, :]


def _combine_kernel(row_ref, out0_ref, out_ref, tail_v, sem, *, b, l):
    del out0_ref
    for bb in range(b):
        pltpu.make_async_copy(
            out_ref.at[bb, pl.ds(l - 8, 8)], tail_v, sem).start()
        pltpu.make_async_copy(
            out_ref.at[bb, pl.ds(l - 8, 8)], tail_v, sem).wait()
        tail_v[7:8, :] = row_ref[0:1, :]
        pltpu.make_async_copy(
            tail_v, out_ref.at[bb, pl.ds(l - 8, 8)], sem).start()
        pltpu.make_async_copy(
            tail_v, out_ref.at[bb, pl.ds(l - 8, 8)], sem).wait()


def kernel(idea, token, emb):
    import functools

    b, l, d = idea.shape
    lout = min(_CONTEXT_WINDOW, l + 1)
    if lout == l + 1:
        idea = jnp.concatenate([jnp.zeros((b, 1, d), idea.dtype), idea], axis=1)
        l = lout
    r = 1024 if l % 1024 == 0 else l
    nb = l // r
    emb_row = _emb_row_sc(emb, token, d)
    out0 = pl.pallas_call(
        _shift_copy_kernel,
        grid=(b, nb),
        in_specs=[pl.BlockSpec((1, r, d), lambda bb, j: (bb, nb - 1 - j, 0))],
        out_specs=pl.BlockSpec((1, r, d), lambda bb, j: (bb, nb - 1 - j, 0)),
        scratch_shapes=[pltpu.VMEM((1, d), idea.dtype)],
        out_shape=jax.ShapeDtypeStruct((b, l, d), idea.dtype),
        compiler_params=pltpu.CompilerParams(
            dimension_semantics=("parallel", "arbitrary"),
            vmem_limit_bytes=100 * 1024 * 1024,
        ),
    )(idea)
    out = pl.pallas_call(
        functools.partial(_combine_kernel, b=b, l=l),
        in_specs=[
            pl.BlockSpec(memory_space=pltpu.MemorySpace.VMEM),
            pl.BlockSpec(memory_space=pl.ANY),
        ],
        out_specs=pl.BlockSpec(memory_space=pl.ANY),
        scratch_shapes=[pltpu.VMEM((8, d), idea.dtype),
                        pltpu.SemaphoreType.DMA],
        out_shape=jax.ShapeDtypeStruct((b, l, d), idea.dtype),
        input_output_aliases={1: 0},
    )(emb_row, out0)
    return out


# speedup vs baseline: 1.2437x; 1.2377x over previous
import functools

import jax
import jax.numpy as jnp
from jax.experimental import pallas as pl
from jax.experimental.pallas import tpu as pltpu

_CONTEXT_WINDOW = 4096


def _loopback_kernel(tok_ref, idea_ref, emb_ref, out_ref, carry_ref, *, nb):
    j = pl.program_id(1)
    r = idea_ref.shape[1]
    out_ref[0, 0:r - 1, :] = idea_ref[0, 1:r, :]

    @pl.when(j == 0)
    def _():
        sub = tok_ref[0] % 8
        vals = emb_ref[...]
        rows = jax.lax.broadcasted_iota(jnp.int32, vals.shape, 0)
        row = jnp.sum(jnp.where(rows == sub, vals, 0.0), axis=0, keepdims=True)
        out_ref[0, r - 1:r, :] = row

    @pl.when(j != 0)
    def _():
        out_ref[0, r - 1:r, :] = carry_ref[...]

    carry_ref[...] = idea_ref[0, 0:1, :]


def kernel(idea, token, emb):
    b, l, d = idea.shape
    lout = min(_CONTEXT_WINDOW, l + 1)
    if lout == l + 1:
        idea = jnp.concatenate([jnp.zeros((b, 1, d), idea.dtype), idea], axis=1)
        l = lout
    r = 512 if l % 512 == 0 else l
    nb = l // r
    tok = jnp.asarray(token, jnp.int32).reshape(1)
    grid_spec = pltpu.PrefetchScalarGridSpec(
        num_scalar_prefetch=1,
        grid=(b, nb),
        in_specs=[
            pl.BlockSpec((1, r, d), lambda bb, j, tok: (bb, nb - 1 - j, 0)),
            pl.BlockSpec((8, d), lambda bb, j, tok: (tok[0] // 8, 0)),
        ],
        out_specs=pl.BlockSpec((1, r, d), lambda bb, j, tok: (bb, nb - 1 - j, 0)),
        scratch_shapes=[pltpu.VMEM((1, d), idea.dtype)],
    )
    out = pl.pallas_call(
        functools.partial(_loopback_kernel, nb=nb),
        grid_spec=grid_spec,
        out_shape=jax.ShapeDtypeStruct((b, l, d), idea.dtype),
        compiler_params=pltpu.CompilerParams(
            dimension_semantics=("parallel", "arbitrary"),
            vmem_limit_bytes=100 * 1024 * 1024,
        ),
    )(tok, idea, emb)
    return out


# speedup vs baseline: 1.2697x; 1.0209x over previous
import functools

import jax
import jax.numpy as jnp
from jax.experimental import pallas as pl
from jax.experimental.pallas import tpu as pltpu

_CONTEXT_WINDOW = 4096


def _loopback_kernel(tok_ref, idea_ref, emb_ref, out_ref):
    r = idea_ref.shape[1]
    out_ref[0, 0:r - 1, :] = idea_ref[0, 1:r, :]
    sub = tok_ref[0] % 8
    vals = emb_ref[...]
    rows = jax.lax.broadcasted_iota(jnp.int32, vals.shape, 0)
    row = jnp.sum(jnp.where(rows == sub, vals, 0.0), axis=0, keepdims=True)
    out_ref[0, r - 1:r, :] = row


def kernel(idea, token, emb):
    b, l, d = idea.shape
    lout = min(_CONTEXT_WINDOW, l + 1)
    if lout == l + 1:
        idea = jnp.concatenate([jnp.zeros((b, 1, d), idea.dtype), idea], axis=1)
        l = lout
    dc = 512 if d % 512 == 0 else d
    nd = d // dc
    tok = jnp.asarray(token, jnp.int32).reshape(1)
    grid_spec = pltpu.PrefetchScalarGridSpec(
        num_scalar_prefetch=1,
        grid=(b, nd),
        in_specs=[
            pl.BlockSpec((1, l, dc), lambda bb, j, tok: (bb, 0, j)),
            pl.BlockSpec((8, dc), lambda bb, j, tok: (tok[0] // 8, j)),
        ],
        out_specs=pl.BlockSpec((1, l, dc), lambda bb, j, tok: (bb, 0, j)),
    )
    out = pl.pallas_call(
        _loopback_kernel,
        grid_spec=grid_spec,
        out_shape=jax.ShapeDtypeStruct((b, l, d), idea.dtype),
        compiler_params=pltpu.CompilerParams(
            dimension_semantics=("parallel", "parallel"),
            vmem_limit_bytes=100 * 1024 * 1024,
        ),
    )(tok, idea, emb)
    return out
